# Initial kernel scaffold; baseline (speedup 1.0000x reference)
import functools

import jax
import jax.numpy as jnp
from jax import lax
from jax.experimental import pallas as pl
from jax.experimental.pallas import tpu as pltpu
from jax.experimental.pallas import tpu_sc as plsc

_VOCAB = 1000000
_EMBED = 64
_B = 16384
_L = 50
_N = _B * _L

_NC = 2
_NS = 16
_NW = _NC * _NS

_BLK = 128
_K = 5
_GROUP_ROWS = _K * _BLK
_ROWS_PER_W = _N // _NW
_BLKS_PER_W = _ROWS_PER_W // _BLK
_GROUPS = _ROWS_PER_W // _GROUP_ROWS


def _body(idx_hbm, table_hbm, out_hbm, idx_v, rows_v, gs0, gs1, os0, os1):
  gsem = (gs0, gs1)
  osem = (os0, os1)
  wid = lax.axis_index("s") * _NC + lax.axis_index("c")
  base_blk = wid * _BLKS_PER_W

  def load_idx(g, s):
    pltpu.sync_copy(idx_hbm.at[pl.ds(base_blk + g * _K, _K)], idx_v.at[s])

  def fire_gathers(s):
    for j in range(_K):
      pltpu.async_copy(table_hbm.at[idx_v.at[s, j]], rows_v.at[s, j], gsem[s])

  def drain_gathers(s):
    for j in range(_K):
      pltpu.make_async_copy(
          table_hbm.at[idx_v.at[s, j]], rows_v.at[s, j], gsem[s]).wait()

  def out_slice(g, j):
    return out_hbm.at[pl.ds((base_blk + g * _K + j) * _BLK, _BLK)]

  def fire_writes(g, s):
    for j in range(_K):
      pltpu.async_copy(rows_v.at[s, j], out_slice(g, j), osem[s])

  def drain_writes(g, s):
    for j in range(_K):
      pltpu.make_async_copy(rows_v.at[s, j], out_slice(g, j), osem[s]).wait()

  def step(g, s, drainw):
    if drainw:
      drain_writes(g - 2, s)
    load_idx(g, s)
    fire_gathers(s)
    drain_gathers(1 - s)
    fire_writes(g - 1, 1 - s)

  load_idx(0, 0)
  fire_gathers(0)
  step(1, 1, drainw=False)

  @pl.loop(2, _GROUPS - 2, step=2)
  def _(gbase):
    for b in range(2):
      step(gbase + b, b, drainw=True)

  step(_GROUPS - 2, 0, drainw=True)
  step(_GROUPS - 1, 1, drainw=True)
  drain_gathers(1)
  fire_writes(_GROUPS - 1, 1)
  drain_writes(_GROUPS - 2, 0)
  drain_writes(_GROUPS - 1, 1)


@jax.jit
def _embed(idx2, table):
  mesh = plsc.VectorSubcoreMesh(core_axis_name="c", subcore_axis_name="s")
  return pl.kernel(
      _body,
      out_type=jax.ShapeDtypeStruct((_N, _EMBED), jnp.float32),
      mesh=mesh,
      scratch_types=[
          pltpu.VMEM((2, _K, _BLK), jnp.int32),
          pltpu.VMEM((2, _K, _BLK, _EMBED), jnp.float32),
          pltpu.SemaphoreType.DMA,
          pltpu.SemaphoreType.DMA,
          pltpu.SemaphoreType.DMA,
          pltpu.SemaphoreType.DMA,
      ],
  )(idx2, table)


def kernel(tokens, table):
  idx2 = tokens.astype(jnp.int32).reshape(_N // _BLK, _BLK)
  out = _embed(idx2, table)
  return out.reshape(_B, _L, _EMBED)


# speedup vs baseline: 1.8748x; 1.8748x over previous
import jax
import jax.numpy as jnp
from jax import lax
from jax.experimental import pallas as pl
from jax.experimental.pallas import tpu as pltpu
from jax.experimental.pallas import tpu_sc as plsc

_VOCAB = 1000000
_EMBED = 64
_B = 16384
_L = 50
_N = _B * _L

_NC = 2
_NS = 16
_NW = _NC * _NS

_BLK = 128
_K = 4
_GROUP_ROWS = _K * _BLK
_ROWS_PER_W = _N // _NW
_BLKS_PER_W = _ROWS_PER_W // _BLK
_CHUNKS_PER_W = _BLKS_PER_W // 8
_GROUPS = _ROWS_PER_W // _GROUP_ROWS


def _body(idx_hbm, table_hbm, out_hbm, idx_v, rows_v, gs0, gs1, os0, os1):
  gsem = (gs0, gs1)
  osem = (os0, os1)
  wid = lax.axis_index("s") * _NC + lax.axis_index("c")
  row0 = wid * _ROWS_PER_W

  def idx_ref(g, j):
    bb = g * _K + j
    return idx_v.at[bb >> 3, bb & 7]

  def fire_gathers(g, s):
    for j in range(_K):
      pltpu.async_copy(table_hbm.at[idx_ref(g, j)], rows_v.at[s, j], gsem[s])

  def drain_gathers(g, s):
    for j in range(_K):
      pltpu.make_async_copy(
          table_hbm.at[idx_ref(g, j)], rows_v.at[s, j], gsem[s]).wait()

  def out_slice(g, j):
    return out_hbm.at[pl.ds(row0 + g * _GROUP_ROWS + j * _BLK, _BLK)]

  def fire_writes(g, s):
    for j in range(_K):
      pltpu.async_copy(rows_v.at[s, j], out_slice(g, j), osem[s])

  def drain_writes(g, s):
    for j in range(_K):
      pltpu.make_async_copy(rows_v.at[s, j], out_slice(g, j), osem[s]).wait()

  def step(g, s, drainw):
    if drainw:
      drain_writes(g - 2, s)
    fire_gathers(g, s)
    drain_gathers(g - 1, 1 - s)
    fire_writes(g - 1, 1 - s)

  pltpu.sync_copy(idx_hbm.at[pl.ds(wid * _CHUNKS_PER_W, _CHUNKS_PER_W)], idx_v)

  fire_gathers(0, 0)
  step(1, 1, drainw=False)

  @pl.loop(2, _GROUPS - 2, step=2)
  def _(gbase):
    for b in range(2):
      step(gbase + b, b, drainw=True)

  step(_GROUPS - 2, 0, drainw=True)
  step(_GROUPS - 1, 1, drainw=True)
  drain_gathers(_GROUPS - 1, 1)
  fire_writes(_GROUPS - 1, 1)
  drain_writes(_GROUPS - 2, 0)
  drain_writes(_GROUPS - 1, 1)


@jax.jit
def _embed(idx3, table):
  mesh = plsc.VectorSubcoreMesh(core_axis_name="c", subcore_axis_name="s")
  return pl.kernel(
      _body,
      out_type=jax.ShapeDtypeStruct((_N, _EMBED), jnp.float32),
      mesh=mesh,
      compiler_params=pltpu.CompilerParams(use_tc_tiling_on_sc=False),
      scratch_types=[
          pltpu.VMEM((_CHUNKS_PER_W, 8, _BLK), jnp.int32),
          pltpu.VMEM((2, _K, _BLK, _EMBED), jnp.float32),
          pltpu.SemaphoreType.DMA,
          pltpu.SemaphoreType.DMA,
          pltpu.SemaphoreType.DMA,
          pltpu.SemaphoreType.DMA,
      ],
  )(idx3, table)


def kernel(tokens, table):
  idx3 = tokens.astype(jnp.int32).reshape(_N // (8 * _BLK), 8, _BLK)
  out = _embed(idx3, table)
  return out.reshape(_B, _L, _EMBED)
